# baseline (speedup 1.0000x reference)
import jax
import jax.numpy as jnp
from jax.experimental import pallas as pl
from jax.experimental.pallas import tpu as pltpu

GAMMA = 12.0
EMB_INIT = 0.109375
PI_REF = 3.1415926235897933
SCALE = PI_REF / EMB_INIT

N_EDGES = 320000
D = 128
EDGE_BLK = 512
N_BLK = N_EDGES // EDGE_BLK

_INV_PI = 0.3183098861837907
_MAGIC = 12582912.0
_PI_HI = 3.140625
_PI_LO = 9.67653589793e-04
_SA = 0.9999389601350983
_SB = -0.16614390484274005
_SC = 0.007689812487933795


def _score_body(src_ref, dst_ref, rel_ref, mod_ref, out_ref):
    x = (src_ref[...] + rel_ref[...] - dst_ref[...]) * SCALE
    n = jnp.round(x * _INV_PI)
    z = (x - n * _PI_HI) - n * _PI_LO
    z2 = z * z
    p = (_SC * z2 + _SB) * z2 + _SA
    abs_sin = jnp.abs(z) * p
    s = GAMMA - jnp.sum(abs_sin, axis=-1)
    out_ref[...] = s * mod_ref[0, 0]


def kernel(node_emb, rel_emb, edge_index, modulus):
    src = jnp.take(node_emb, edge_index[0], axis=0)
    dst = jnp.take(node_emb, edge_index[1], axis=0)

    grid_spec = pl.GridSpec(
        grid=(N_BLK,),
        in_specs=[
            pl.BlockSpec((EDGE_BLK, D), lambda i: (i, 0)),
            pl.BlockSpec((EDGE_BLK, D), lambda i: (i, 0)),
            pl.BlockSpec((EDGE_BLK, D), lambda i: (i, 0)),
            pl.BlockSpec(memory_space=pltpu.SMEM),
        ],
        out_specs=pl.BlockSpec((EDGE_BLK,), lambda i: (i,)),
    )
    out = pl.pallas_call(
        _score_body,
        grid_spec=grid_spec,
        out_shape=jax.ShapeDtypeStruct((N_EDGES,), jnp.float32),
    )(src, dst, rel_emb, modulus)
    return out


# speedup vs baseline: 1.2841x; 1.2841x over previous
import functools

import jax
import jax.numpy as jnp
from jax import lax
from jax.experimental import pallas as pl
from jax.experimental.pallas import tpu as pltpu
from jax.experimental.pallas import tpu_sc as plsc

GAMMA = 12.0
EMB_INIT = 0.109375
PI_REF = 3.1415926235897933
SCALE = PI_REF / EMB_INIT

N_NODES = 10000
N_EDGES = 320000
D = 128
EDGE_BLK = 512
N_BLK = N_EDGES // EDGE_BLK

NC = 2
NS = 16
NW = NC * NS
G = 128
N_GATHER = 2 * N_EDGES
ROWS_PER_W = 20480
N_PAD = NW * ROWS_PER_W
CHUNKS_PER_W = ROWS_PER_W // G

_INV_PI = 0.3183098861837907
_PI_HI = 3.140625
_PI_LO = 9.67653589793e-04
_SA = 0.9999389601350983
_SB = -0.16614390484274005
_SC = 0.007689812487933795


def _gather_body(table_hbm, idx_hbm, out_hbm, idx_v, rows_v, sem):
    wid = lax.axis_index("s") * NC + lax.axis_index("c")
    base = wid * ROWS_PER_W
    pltpu.sync_copy(idx_hbm.at[pl.ds(base, ROWS_PER_W)], idx_v)

    def step(j, carry):
        row0 = base + j * G
        pltpu.async_copy(
            table_hbm.at[idx_v.at[pl.ds(j * G, G)]], rows_v, sem
        ).wait()
        pltpu.sync_copy(rows_v, out_hbm.at[pl.ds(row0, G)])
        return carry

    lax.fori_loop(0, CHUNKS_PER_W, step, 0)


@functools.partial(
    pl.kernel,
    mesh=plsc.VectorSubcoreMesh(core_axis_name="c", subcore_axis_name="s"),
    out_type=jax.ShapeDtypeStruct((N_PAD, D), jnp.float32),
    scratch_types=[
        pltpu.VMEM((ROWS_PER_W,), jnp.int32),
        pltpu.VMEM((G, D), jnp.float32),
        pltpu.SemaphoreType.DMA,
    ],
)
def _sc_gather(table_hbm, idx_hbm, out_hbm, idx_v, rows_v, sem):
    _gather_body(table_hbm, idx_hbm, out_hbm, idx_v, rows_v, sem)


def _score_body(src_ref, dst_ref, rel_ref, mod_ref, out_ref):
    x = (src_ref[...] + rel_ref[...] - dst_ref[...]) * SCALE
    n = jnp.round(x * _INV_PI)
    z = (x - n * _PI_HI) - n * _PI_LO
    z2 = z * z
    p = (_SC * z2 + _SB) * z2 + _SA
    abs_sin = jnp.abs(z) * p
    s = GAMMA - jnp.sum(abs_sin, axis=-1)
    out_ref[...] = s * mod_ref[0, 0]


def kernel(node_emb, rel_emb, edge_index, modulus):
    idx_flat = edge_index.reshape(N_GATHER)
    idx_pad = jnp.concatenate(
        [idx_flat, jnp.zeros((N_PAD - N_GATHER,), jnp.int32)]
    )
    gathered = _sc_gather(node_emb, idx_pad)

    dst_off = N_EDGES // EDGE_BLK
    grid_spec = pl.GridSpec(
        grid=(N_BLK,),
        in_specs=[
            pl.BlockSpec((EDGE_BLK, D), lambda i: (i, 0)),
            pl.BlockSpec((EDGE_BLK, D), lambda i: (i + dst_off, 0)),
            pl.BlockSpec((EDGE_BLK, D), lambda i: (i, 0)),
            pl.BlockSpec(memory_space=pltpu.SMEM),
        ],
        out_specs=pl.BlockSpec((EDGE_BLK,), lambda i: (i,)),
    )
    return pl.pallas_call(
        _score_body,
        grid_spec=grid_spec,
        out_shape=jax.ShapeDtypeStruct((N_EDGES,), jnp.float32),
    )(gathered, gathered, rel_emb, modulus)


# speedup vs baseline: 1.3506x; 1.0518x over previous
import functools

import jax
import jax.numpy as jnp
from jax import lax
from jax.experimental import pallas as pl
from jax.experimental.pallas import tpu as pltpu
from jax.experimental.pallas import tpu_sc as plsc

GAMMA = 12.0
EMB_INIT = 0.109375
PI_REF = 3.1415926235897933
SCALE = PI_REF / EMB_INIT

N_NODES = 10000
N_EDGES = 320000
D = 128
EDGE_BLK = 512
N_BLK = N_EDGES // EDGE_BLK

NC = 2
NS = 16
NW = NC * NS
G = 128
N_GATHER = 2 * N_EDGES
ROWS_PER_W = 20480
N_PAD = NW * ROWS_PER_W
CHUNKS_PER_W = ROWS_PER_W // G

_INV_PI = 0.3183098861837907
_PI_HI = 3.140625
_PI_LO = 9.67653589793e-04
_SA = 0.9999389601350983
_SB = -0.16614390484274005
_SC = 0.007689812487933795


def _gather_body(table_hbm, idx_hbm, out_hbm, idx_v, rows0, rows1,
                 sg0, sg1, sw0, sw1):
    wid = lax.axis_index("s") * NC + lax.axis_index("c")
    base = wid * ROWS_PER_W
    pltpu.sync_copy(idx_hbm.at[pl.ds(base, ROWS_PER_W)], idx_v)

    bufs = (rows0, rows1)
    gsems = (sg0, sg1)
    wsems = (sw0, sw1)

    def start_gather(j, k):
        pltpu.async_copy(
            table_hbm.at[idx_v.at[pl.ds(j * G, G)]], bufs[k], gsems[k]
        )

    def wait_gather(k):
        pltpu.make_async_copy(
            out_hbm.at[pl.ds(0, G)], bufs[k], gsems[k]
        ).wait()

    def start_write(j, k):
        pltpu.async_copy(bufs[k], out_hbm.at[pl.ds(base + j * G, G)], wsems[k])

    def wait_write(k):
        pltpu.make_async_copy(
            bufs[k], out_hbm.at[pl.ds(0, G)], wsems[k]
        ).wait()

    start_gather(0, 0)

    def step2(i, carry):
        for k in (0, 1):
            j = i * 2 + k
            k2 = 1 - k
            wait_gather(k)
            start_write(j, k)
            @pl.when(j >= 1)
            def _():
                wait_write(k2)

            @pl.when(j < CHUNKS_PER_W - 1)
            def _():
                start_gather(j + 1, k2)

        return carry

    lax.fori_loop(0, CHUNKS_PER_W // 2, step2, 0)
    wait_write(1)


@functools.partial(
    pl.kernel,
    mesh=plsc.VectorSubcoreMesh(core_axis_name="c", subcore_axis_name="s"),
    out_type=jax.ShapeDtypeStruct((N_PAD, D), jnp.float32),
    scratch_types=[
        pltpu.VMEM((ROWS_PER_W,), jnp.int32),
        pltpu.VMEM((G, D), jnp.float32),
        pltpu.VMEM((G, D), jnp.float32),
        pltpu.SemaphoreType.DMA,
        pltpu.SemaphoreType.DMA,
        pltpu.SemaphoreType.DMA,
        pltpu.SemaphoreType.DMA,
    ],
)
def _sc_gather(table_hbm, idx_hbm, out_hbm, idx_v, rows0, rows1,
               sg0, sg1, sw0, sw1):
    _gather_body(table_hbm, idx_hbm, out_hbm, idx_v, rows0, rows1,
                 sg0, sg1, sw0, sw1)


def _score_body(src_ref, dst_ref, rel_ref, mod_ref, out_ref):
    x = (src_ref[...] + rel_ref[...] - dst_ref[...]) * SCALE
    n = jnp.round(x * _INV_PI)
    z = (x - n * _PI_HI) - n * _PI_LO
    z2 = z * z
    p = (_SC * z2 + _SB) * z2 + _SA
    abs_sin = jnp.abs(z) * p
    s = GAMMA - jnp.sum(abs_sin, axis=-1)
    out_ref[...] = s * mod_ref[0, 0]


def kernel(node_emb, rel_emb, edge_index, modulus):
    idx_flat = edge_index.reshape(N_GATHER)
    idx_pad = jnp.concatenate(
        [idx_flat, jnp.zeros((N_PAD - N_GATHER,), jnp.int32)]
    )
    gathered = _sc_gather(node_emb, idx_pad)

    dst_off = N_EDGES // EDGE_BLK
    grid_spec = pl.GridSpec(
        grid=(N_BLK,),
        in_specs=[
            pl.BlockSpec((EDGE_BLK, D), lambda i: (i, 0)),
            pl.BlockSpec((EDGE_BLK, D), lambda i: (i + dst_off, 0)),
            pl.BlockSpec((EDGE_BLK, D), lambda i: (i, 0)),
            pl.BlockSpec(memory_space=pltpu.SMEM),
        ],
        out_specs=pl.BlockSpec((EDGE_BLK,), lambda i: (i,)),
    )
    return pl.pallas_call(
        _score_body,
        grid_spec=grid_spec,
        out_shape=jax.ShapeDtypeStruct((N_EDGES,), jnp.float32),
    )(gathered, gathered, rel_emb, modulus)


# speedup vs baseline: 3.2568x; 2.4114x over previous
import functools

import jax
import jax.numpy as jnp
from jax import lax
from jax.experimental import pallas as pl
from jax.experimental.pallas import tpu as pltpu
from jax.experimental.pallas import tpu_sc as plsc

GAMMA = 12.0
EMB_INIT = 0.109375
PI_REF = 3.1415926235897933
SCALE = PI_REF / EMB_INIT

N_NODES = 10000
N_EDGES = 320000
D = 128
EDGE_BLK = 512
N_BLK = N_EDGES // EDGE_BLK

NC = 2
NS = 16
NW = NC * NS
G = 128
N_GATHER = 2 * N_EDGES
ROWS_PER_W = 20480
N_PAD = NW * ROWS_PER_W
CHUNKS_PER_W = ROWS_PER_W // G

_INV_PI = 0.3183098861837907
_PI_HI = 3.140625
_PI_LO = 9.67653589793e-04
_SA = 0.9999389601350983
_SB = -0.16614390484274005
_SC = 0.007689812487933795


def _gather_body(table_hbm, idx_hbm, out_hbm, table_sp, idx0, idx1,
                 rows0, rows1, si0, si1, sg0, sg1, sw0, sw1):
    sid = lax.axis_index("s")
    wid = sid * NC + lax.axis_index("c")
    base = wid * ROWS_PER_W

    @pl.when(sid == 0)
    def _():
        pltpu.sync_copy(table_hbm, table_sp)

    plsc.subcore_barrier()

    ibufs = (idx0, idx1)
    bufs = (rows0, rows1)
    isems = (si0, si1)
    gsems = (sg0, sg1)
    wsems = (sw0, sw1)

    def start_idx(j, k):
        pltpu.async_copy(
            idx_hbm.at[pl.ds(base + j * G, G)], ibufs[k], isems[k]
        )

    def wait_idx(k):
        pltpu.make_async_copy(
            idx_hbm.at[pl.ds(0, G)], ibufs[k], isems[k]
        ).wait()

    def start_gather(j, k):
        pltpu.async_copy(table_sp.at[ibufs[k]], bufs[k], gsems[k])

    def wait_gather(k):
        pltpu.make_async_copy(
            out_hbm.at[pl.ds(0, G)], bufs[k], gsems[k]
        ).wait()

    def start_write(j, k):
        pltpu.async_copy(bufs[k], out_hbm.at[pl.ds(base + j * G, G)], wsems[k])

    def wait_write(k):
        pltpu.make_async_copy(
            bufs[k], out_hbm.at[pl.ds(0, G)], wsems[k]
        ).wait()

    start_idx(0, 0)
    start_idx(1, 1)
    wait_idx(0)
    start_gather(0, 0)

    def step2(i, carry):
        for k in (0, 1):
            j = i * 2 + k
            k2 = 1 - k
            wait_gather(k)
            start_write(j, k)

            @pl.when(j < CHUNKS_PER_W - 2)
            def _():
                start_idx(j + 2, k)

            @pl.when(j < CHUNKS_PER_W - 1)
            def _():
                wait_idx(k2)
                @pl.when(j >= 1)
                def _():
                    wait_write(k2)

                start_gather(j + 1, k2)

        return carry

    lax.fori_loop(0, CHUNKS_PER_W // 2, step2, 0)
    wait_write(1)


@functools.partial(
    pl.kernel,
    mesh=plsc.VectorSubcoreMesh(core_axis_name="c", subcore_axis_name="s"),
    out_type=jax.ShapeDtypeStruct((N_PAD, D), jnp.float32),
    scratch_types=[
        pltpu.VMEM_SHARED((N_NODES, D), jnp.float32),
        pltpu.VMEM((G,), jnp.int32),
        pltpu.VMEM((G,), jnp.int32),
        pltpu.VMEM((G, D), jnp.float32),
        pltpu.VMEM((G, D), jnp.float32),
        pltpu.SemaphoreType.DMA,
        pltpu.SemaphoreType.DMA,
        pltpu.SemaphoreType.DMA,
        pltpu.SemaphoreType.DMA,
        pltpu.SemaphoreType.DMA,
        pltpu.SemaphoreType.DMA,
    ],
)
def _sc_gather(table_hbm, idx_hbm, out_hbm, table_sp, idx0, idx1,
               rows0, rows1, si0, si1, sg0, sg1, sw0, sw1):
    _gather_body(table_hbm, idx_hbm, out_hbm, table_sp, idx0, idx1,
                 rows0, rows1, si0, si1, sg0, sg1, sw0, sw1)


def _score_body(src_ref, dst_ref, rel_ref, mod_ref, out_ref):
    x = (src_ref[...] + rel_ref[...] - dst_ref[...]) * SCALE
    n = jnp.round(x * _INV_PI)
    z = (x - n * _PI_HI) - n * _PI_LO
    z2 = z * z
    p = (_SC * z2 + _SB) * z2 + _SA
    abs_sin = jnp.abs(z) * p
    s = GAMMA - jnp.sum(abs_sin, axis=-1)
    out_ref[...] = s * mod_ref[0, 0]


def kernel(node_emb, rel_emb, edge_index, modulus):
    idx_flat = edge_index.reshape(N_GATHER)
    idx_pad = jnp.concatenate(
        [idx_flat, jnp.zeros((N_PAD - N_GATHER,), jnp.int32)]
    )
    gathered = _sc_gather(node_emb, idx_pad)

    dst_off = N_EDGES // EDGE_BLK
    grid_spec = pl.GridSpec(
        grid=(N_BLK,),
        in_specs=[
            pl.BlockSpec((EDGE_BLK, D), lambda i: (i, 0)),
            pl.BlockSpec((EDGE_BLK, D), lambda i: (i + dst_off, 0)),
            pl.BlockSpec((EDGE_BLK, D), lambda i: (i, 0)),
            pl.BlockSpec(memory_space=pltpu.SMEM),
        ],
        out_specs=pl.BlockSpec((EDGE_BLK,), lambda i: (i,)),
    )
    return pl.pallas_call(
        _score_body,
        grid_spec=grid_spec,
        out_shape=jax.ShapeDtypeStruct((N_EDGES,), jnp.float32),
    )(gathered, gathered, rel_emb, modulus)


# speedup vs baseline: 5.2859x; 1.6230x over previous
import functools

import jax
import jax.numpy as jnp
from jax import lax
from jax.experimental import pallas as pl
from jax.experimental.pallas import tpu as pltpu
from jax.experimental.pallas import tpu_sc as plsc

GAMMA = 12.0
EMB_INIT = 0.109375
PI_REF = 3.1415926235897933
SCALE = PI_REF / EMB_INIT

N_NODES = 10000
N_EDGES = 320000
D = 128
EDGE_BLK = 2048
N_EDGES_PAD = 327680
N_BLK = N_EDGES_PAD // EDGE_BLK
LAST_REL_BLK = (N_EDGES - 1) // EDGE_BLK

NC = 2
NS = 16
NW = NC * NS
G = 128
N_PAD = 2 * N_EDGES_PAD

P = 4
BLK_P = N_BLK // P
ROWS_P = N_PAD // P
ROWS_PER_W = ROWS_P // NW
CHUNKS_PER_W = ROWS_PER_W // G

_INV_PI = 0.3183098861837907
_PI_HI = 3.140625
_PI_LO = 9.67653589793e-04
_SA = 0.9999389601350983
_SB = -0.16614390484274005
_SC = 0.007689812487933795


def _gather_body(part, table_hbm, idx_hbm, out_hbm, table_sp, idx0, idx1,
                 rows0, rows1, si0, si1, sg0, sg1, sw0, sw1):
    sid = lax.axis_index("s")
    wid = sid * NC + lax.axis_index("c")
    base = wid * ROWS_PER_W
    idx_base = part * ROWS_P + base

    @pl.when(sid == 0)
    def _():
        pltpu.sync_copy(table_hbm, table_sp)

    plsc.subcore_barrier()

    ibufs = (idx0, idx1)
    bufs = (rows0, rows1)
    isems = (si0, si1)
    gsems = (sg0, sg1)
    wsems = (sw0, sw1)

    def start_idx(j, k):
        pltpu.async_copy(
            idx_hbm.at[pl.ds(idx_base + j * G, G)], ibufs[k], isems[k]
        )

    def wait_idx(k):
        pltpu.make_async_copy(
            idx_hbm.at[pl.ds(0, G)], ibufs[k], isems[k]
        ).wait()

    def start_gather(j, k):
        pltpu.async_copy(table_sp.at[ibufs[k]], bufs[k], gsems[k])

    def wait_gather(k):
        pltpu.make_async_copy(
            out_hbm.at[pl.ds(0, G)], bufs[k], gsems[k]
        ).wait()

    def start_write(j, k):
        pltpu.async_copy(bufs[k], out_hbm.at[pl.ds(base + j * G, G)], wsems[k])

    def wait_write(k):
        pltpu.make_async_copy(
            bufs[k], out_hbm.at[pl.ds(0, G)], wsems[k]
        ).wait()

    start_idx(0, 0)
    start_idx(1, 1)
    wait_idx(0)
    start_gather(0, 0)

    def step2(i, carry):
        for k in (0, 1):
            j = i * 2 + k
            k2 = 1 - k
            wait_gather(k)
            start_write(j, k)

            @pl.when(j < CHUNKS_PER_W - 2)
            def _():
                start_idx(j + 2, k)

            @pl.when(j < CHUNKS_PER_W - 1)
            def _():
                wait_idx(k2)
                @pl.when(j >= 1)
                def _():
                    wait_write(k2)

                start_gather(j + 1, k2)

        return carry

    lax.fori_loop(0, CHUNKS_PER_W // 2, step2, 0)
    wait_write(1)


def _make_sc_gather(part):
    @functools.partial(
        pl.kernel,
        mesh=plsc.VectorSubcoreMesh(core_axis_name="c", subcore_axis_name="s"),
        out_type=jax.ShapeDtypeStruct((ROWS_P, D), jnp.float32),
        scratch_types=[
            pltpu.VMEM_SHARED((N_NODES, D), jnp.float32),
            pltpu.VMEM((G,), jnp.int32),
            pltpu.VMEM((G,), jnp.int32),
            pltpu.VMEM((G, D), jnp.float32),
            pltpu.VMEM((G, D), jnp.float32),
            pltpu.SemaphoreType.DMA,
            pltpu.SemaphoreType.DMA,
            pltpu.SemaphoreType.DMA,
            pltpu.SemaphoreType.DMA,
            pltpu.SemaphoreType.DMA,
            pltpu.SemaphoreType.DMA,
        ],
    )
    def _sc_gather(table_hbm, idx_hbm, out_hbm, *rest):
        _gather_body(part, table_hbm, idx_hbm, out_hbm, *rest)

    return _sc_gather


_SC_GATHERS = [_make_sc_gather(p) for p in range(P)]


def _score_body(gath_ref, rel_ref, mod_ref, out_ref):
    src = gath_ref[0:EDGE_BLK, :]
    dst = gath_ref[EDGE_BLK : 2 * EDGE_BLK, :]
    x = (src + rel_ref[...] - dst) * SCALE
    n = jnp.round(x * _INV_PI)
    z = (x - n * _PI_HI) - n * _PI_LO
    z2 = z * z
    p = (_SC * z2 + _SB) * z2 + _SA
    abs_sin = jnp.abs(z) * p
    s = GAMMA - jnp.sum(abs_sin, axis=-1)
    out_ref[...] = s * mod_ref[0, 0]


def _score_part(part, gathered, rel_emb, modulus):
    rel_map = functools.partial(
        lambda p, i: (jnp.minimum(p * BLK_P + i, LAST_REL_BLK), 0), part
    )
    grid_spec = pl.GridSpec(
        grid=(BLK_P,),
        in_specs=[
            pl.BlockSpec((2 * EDGE_BLK, D), lambda i: (i, 0)),
            pl.BlockSpec((EDGE_BLK, D), rel_map),
            pl.BlockSpec(memory_space=pltpu.SMEM),
        ],
        out_specs=pl.BlockSpec((EDGE_BLK,), lambda i: (i,)),
    )
    return pl.pallas_call(
        _score_body,
        grid_spec=grid_spec,
        out_shape=jax.ShapeDtypeStruct((BLK_P * EDGE_BLK,), jnp.float32),
    )(gathered, rel_emb, modulus)


def kernel(node_emb, rel_emb, edge_index, modulus):
    pad = jnp.zeros((N_EDGES_PAD - N_EDGES,), jnp.int32)
    src_idx = jnp.concatenate([edge_index[0], pad]).reshape(N_BLK, EDGE_BLK)
    dst_idx = jnp.concatenate([edge_index[1], pad]).reshape(N_BLK, EDGE_BLK)
    idx_pad = jnp.stack([src_idx, dst_idx], axis=1).reshape(N_PAD)

    outs = []
    for p in range(P):
        gathered = _SC_GATHERS[p](node_emb, idx_pad)
        outs.append(_score_part(p, gathered, rel_emb, modulus))
    return jnp.concatenate(outs)[:N_EDGES]
